# baseline (speedup 1.0000x reference)
import functools

import jax
import jax.numpy as jnp
from jax import lax
from jax.experimental import pallas as pl
from jax.experimental.pallas import tpu as pltpu
from jax.experimental.pallas import tpu_sc as plsc

_MESH = plsc.VectorSubcoreMesh(core_axis_name="core", subcore_axis_name="subcore")
_NW = 32
_U = 7812
_U_MAIN = _U // _NW
_U_TAIL = _U - _U_MAIN * _NW
_CP = pltpu.CompilerParams(use_tc_tiling_on_sc=True, needs_layout_passes=False)


def _iota16():
  return lax.iota(jnp.int32, 16)


@jax.jit
def _impl(table_t, tail2, idx_t):
  @functools.partial(
      pl.kernel,
      out_type=jax.ShapeDtypeStruct((1000064, 128), jnp.float32),
      mesh=_MESH,
      scratch_types=[
          pltpu.VMEM((64, 128), jnp.float32),
          pltpu.VMEM((64, 128), jnp.float32),
          pltpu.VMEM((64, 129), jnp.float32),
          pltpu.VMEM((128, 128), jnp.float32),
          pltpu.VMEM((128, 128), jnp.float32),
          pltpu.SemaphoreType.DMA,
          pltpu.SemaphoreType.DMA,
          pltpu.SemaphoreType.DMA,
          pltpu.SemaphoreType.DMA,
      ],
      compiler_params=_CP,
  )
  def call_a(tt_hbm, tail_hbm, trm_hbm, in0, in1, ipd, ou0, ou1,
             si0, si1, so0, so1):
    wid = lax.axis_index("subcore") * 2 + lax.axis_index("core")
    inb = (in0, in1)
    oub = (ou0, ou1)
    sin = (si0, si1)
    sou = (so0, so1)

    n_my = jnp.where(wid < _U_TAIL, _U_MAIN + 1, _U_MAIN)

    def start_in(i, s):
      u = i * _NW + wid
      pltpu.async_copy(tt_hbm.at[pl.ds(0, 64), pl.ds(u * 128, 128)],
                       inb[s], sin[s])

    def wait_in(s):
      pltpu.make_async_copy(tt_hbm.at[pl.ds(0, 64), pl.ds(0, 128)],
                            inb[s], sin[s]).wait()

    def start_out(i, s):
      u = i * _NW + wid
      pltpu.async_copy(oub[s], trm_hbm.at[pl.ds(u * 128, 128), pl.ds(0, 128)],
                       sou[s])

    def wait_out(s):
      pltpu.make_async_copy(oub[s],
                            trm_hbm.at[pl.ds(0, 128), pl.ds(0, 128)],
                            sou[s]).wait()

    it = _iota16()
    rows = [it + (16 * q) for q in range(4)]

    def transpose_unit(s):
      @pl.loop(0, 16)
      def _(eb):
        for de in range(4):
          e = eb * 4 + de
          for q in range(8):
            ipd[e, pl.ds(16 * q, 16)] = inb[s][e, pl.ds(16 * q, 16)]

      @pl.loop(0, 16)
      def _(cb):
        for dc in range(8):
          c = cb * 8 + dc
          cv = jnp.full((16,), 0, jnp.int32) + c
          for q in range(4):
            oub[s][c, pl.ds(16 * q, 16)] = plsc.load_gather(
                ipd, [rows[q], cv])

    start_in(0, 0)
    start_in(1, 1)

    @pl.loop(0, (_U_MAIN + 2) // 2)
    def _(i2):
      for s in range(2):
        i = i2 * 2 + s

        @pl.when(i < n_my)
        def _():
          wait_in(s)

          @pl.when(i >= 2)
          def _():
            wait_out(s)

          transpose_unit(s)
          start_out(i, s)

          @pl.when(i + 2 < n_my)
          def _():
            start_in(i + 2, s)

    for s in range(2):
      wait_out(s)

    @pl.when(wid == _NW - 1)
    def _():
      tv = in0.at[pl.ds(0, 32), pl.ds(0, 128)]
      pltpu.sync_copy(tail_hbm, tv)
      for q2 in range(32):
        for half in range(2):
          for q in range(4):
            ou0[2 * q2 + half, pl.ds(16 * q, 16)] = (
                in0[q2, pl.ds(64 * half + 16 * q, 16)])
      pltpu.sync_copy(ou0.at[pl.ds(0, 64)],
                      trm_hbm.at[pl.ds(999936, 64), pl.ds(0, 128)])

  trm = call_a(table_t, tail2)

  @functools.partial(
      pl.kernel,
      out_type=jax.ShapeDtypeStruct((50, 64, 16384), jnp.float32),
      mesh=_MESH,
      scratch_types=[
          pltpu.VMEM((128, 129), jnp.float32),
          pltpu.VMEM((128, 129), jnp.float32),
          pltpu.VMEM((64, 128), jnp.float32),
          pltpu.VMEM((64, 128), jnp.float32),
          pltpu.VMEM((1, 128), jnp.int32),
          pltpu.VMEM((1, 128), jnp.int32),
          pltpu.VMEM((1, 128), jnp.int32),
          pltpu.VMEM((1, 128), jnp.int32),
          pltpu.SemaphoreType.DMA,
          pltpu.SemaphoreType.DMA,
          pltpu.SemaphoreType.DMA,
          pltpu.SemaphoreType.DMA,
          pltpu.SemaphoreType.DMA,
          pltpu.SemaphoreType.DMA,
          pltpu.SemaphoreType.DMA,
          pltpu.SemaphoreType.DMA,
      ],
      compiler_params=_CP,
  )
  def call_b(trm_hbm, it_hbm, out_hbm, ga0, ga1, tr0, tr1,
             iw0, iw1, iw2, iw3,
             sg0, sg1, st0, st1, sw0, sw1, sw2, sw3):
    wid = lax.axis_index("subcore") * 2 + lax.axis_index("core")
    gab = (ga0, ga1)
    trb = (tr0, tr1)
    iwb = (iw0, iw1, iw2, iw3)
    sg = (sg0, sg1)
    st = (st0, st1)
    sw = (sw0, sw1, sw2, sw3)

    def start_idx(w, p):
      h = lax.div(w, 4)
      bg = 4 * wid + lax.rem(w, 4)
      pltpu.async_copy(it_hbm.at[pl.ds(h, 1), pl.ds(bg * 128, 128)],
                       iwb[p], sw[p])

    def wait_idx(p):
      pltpu.make_async_copy(it_hbm.at[pl.ds(0, 1), pl.ds(0, 128)],
                            iwb[p], sw[p]).wait()

    def start_gather(p, s):
      pltpu.async_copy(trm_hbm.at[iwb[p].at[0]],
                       gab[s].at[pl.ds(0, 128), pl.ds(0, 128)], sg[s])

    def wait_gather(p, s):
      pltpu.make_async_copy(trm_hbm.at[iwb[p].at[0]],
                            gab[s].at[pl.ds(0, 128), pl.ds(0, 128)],
                            sg[s]).wait()

    def start_store(w, s):
      h = lax.div(w, 4)
      bg = 4 * wid + lax.rem(w, 4)
      pltpu.async_copy(trb[s],
                       out_hbm.at[h, pl.ds(0, 64), pl.ds(bg * 128, 128)],
                       st[s])

    def wait_store(s):
      pltpu.make_async_copy(trb[s],
                            out_hbm.at[0, pl.ds(0, 64), pl.ds(0, 128)],
                            st[s]).wait()

    it = _iota16()

    def transpose_window(s):
      @pl.loop(0, 8)
      def _(cb):
        rows = it + cb * 16
        for e in range(64):
          ev = jnp.full((16,), e, jnp.int32)
          trb[s][e, pl.ds(cb * 16, 16)] = plsc.load_gather(gab[s], [rows, ev])

    for p in range(4):
      start_idx(p, p)
    wait_idx(0)
    start_gather(0, 0)

    @pl.loop(0, 50)
    def _(w4):
      for ws in range(4):
        w = w4 * 4 + ws
        s = ws % 2
        wait_gather(ws, s)

        @pl.when(w >= 2)
        def _():
          wait_store(s)

        transpose_window(s)
        start_store(w, s)

        @pl.when(w + 4 < 200)
        def _():
          start_idx(w + 4, ws)

        @pl.when(w + 1 < 200)
        def _():
          wait_idx((ws + 1) % 4)
          start_gather((ws + 1) % 4, (ws + 1) % 2)

    for s in range(2):
      wait_store(s)

  return call_b(trm, idx_t)


def kernel(table, type_index):
  tail2 = table[999936:1000000].reshape(32, 128)
  x = _impl(table.T, tail2, type_index.T)
  return x.transpose(2, 0, 1)


# speedup vs baseline: 1.9211x; 1.9211x over previous
import functools

import jax
import jax.numpy as jnp
from jax import lax
from jax.experimental import pallas as pl
from jax.experimental.pallas import tpu as pltpu
from jax.experimental.pallas import tpu_sc as plsc

_MESH = plsc.VectorSubcoreMesh(core_axis_name="core", subcore_axis_name="subcore")
_NW = 32
_U = 7812
_U_MAIN = _U // _NW
_U_TAIL = _U - _U_MAIN * _NW
_CP = pltpu.CompilerParams(use_tc_tiling_on_sc=True, needs_layout_passes=False)


def _iota16():
  return lax.iota(jnp.int32, 16)


@jax.jit
def _impl(table_t, tail2, idx_t):
  @functools.partial(
      pl.kernel,
      out_type=jax.ShapeDtypeStruct((1000064, 128), jnp.float32),
      mesh=_MESH,
      scratch_types=[
          pltpu.VMEM((64, 128), jnp.float32),
          pltpu.VMEM((64, 128), jnp.float32),
          pltpu.VMEM((128, 129), jnp.float32),
          pltpu.VMEM((128, 129), jnp.float32),
          pltpu.VMEM((128,), jnp.int32),
          pltpu.VMEM((128,), jnp.int32),
          pltpu.SemaphoreType.DMA,
          pltpu.SemaphoreType.DMA,
          pltpu.SemaphoreType.DMA,
          pltpu.SemaphoreType.DMA,
      ],
      compiler_params=_CP,
  )
  def call_a(tt_hbm, tail_hbm, trm_hbm, in0, in1, ou0, ou1, rx0, rx1,
             si0, si1, so0, so1):
    wid = lax.axis_index("subcore") * 2 + lax.axis_index("core")
    inb = (in0, in1)
    oub = (ou0, ou1)
    rxb = (rx0, rx1)
    sin = (si0, si1)
    sou = (so0, so1)

    n_my = jnp.where(wid < _U_TAIL, _U_MAIN + 1, _U_MAIN)

    it = _iota16()
    rows = [it + (16 * q) for q in range(8)]

    def start_in(i, s):
      u = i * _NW + wid
      pltpu.async_copy(tt_hbm.at[pl.ds(0, 64), pl.ds(u * 128, 128)],
                       inb[s], sin[s])

    def wait_in(s):
      pltpu.make_async_copy(tt_hbm.at[pl.ds(0, 64), pl.ds(0, 128)],
                            inb[s], sin[s]).wait()

    def ou_slice(s):
      return oub[s].at[pl.ds(0, 128), pl.ds(0, 128)]

    def start_out(i, s):
      u = i * _NW + wid
      for q in range(8):
        rxb[s][pl.ds(16 * q, 16)] = rows[q] + u * 128
      pltpu.async_copy(ou_slice(s), trm_hbm.at[rxb[s]], sou[s])

    def wait_out(s):
      pltpu.make_async_copy(ou_slice(s), trm_hbm.at[rxb[s]], sou[s]).wait()

    def transpose_unit(s):
      @pl.loop(0, 8)
      def _(cb):
        rvec = it + cb * 16
        for e4 in range(16):
          vals = []
          for j in range(4):
            e = e4 * 4 + j
            vals.append(inb[s][e, pl.ds(cb * 16, 16)])
          for j in range(4):
            e = e4 * 4 + j
            plsc.store_scatter(oub[s], [rvec, jnp.full((16,), e, jnp.int32)],
                               vals[j])

    start_in(0, 0)
    start_in(1, 1)

    @pl.loop(0, (_U_MAIN + 2) // 2)
    def _(i2):
      for s in range(2):
        i = i2 * 2 + s

        @pl.when(i < n_my)
        def _():
          wait_in(s)

          @pl.when(i >= 2)
          def _():
            wait_out(s)

          transpose_unit(s)
          start_out(i, s)

          @pl.when(i + 2 < n_my)
          def _():
            start_in(i + 2, s)

    for s in range(2):
      wait_out(s)

    @pl.when(wid == _NW - 1)
    def _():
      tv = in0.at[pl.ds(0, 32), pl.ds(0, 128)]
      pltpu.sync_copy(tail_hbm, tv)
      for q2 in range(32):
        for half in range(2):
          for q in range(4):
            ou0[2 * q2 + half, pl.ds(16 * q, 16)] = (
                in0[q2, pl.ds(64 * half + 16 * q, 16)])
      for q in range(4):
        rx0[pl.ds(16 * q, 16)] = rows[q] + 999936
      pltpu.sync_copy(ou0.at[pl.ds(0, 64), pl.ds(0, 128)],
                      trm_hbm.at[rx0.at[pl.ds(0, 64)]])

  trm = call_a(table_t, tail2)

  @functools.partial(
      pl.kernel,
      out_type=jax.ShapeDtypeStruct((50, 64, 16384), jnp.float32),
      mesh=_MESH,
      scratch_types=[
          pltpu.VMEM((128, 129), jnp.float32),
          pltpu.VMEM((128, 129), jnp.float32),
          pltpu.VMEM((64, 128), jnp.float32),
          pltpu.VMEM((64, 128), jnp.float32),
          pltpu.VMEM((1, 128), jnp.int32),
          pltpu.VMEM((1, 128), jnp.int32),
          pltpu.VMEM((1, 128), jnp.int32),
          pltpu.VMEM((1, 128), jnp.int32),
          pltpu.SemaphoreType.DMA,
          pltpu.SemaphoreType.DMA,
          pltpu.SemaphoreType.DMA,
          pltpu.SemaphoreType.DMA,
          pltpu.SemaphoreType.DMA,
          pltpu.SemaphoreType.DMA,
          pltpu.SemaphoreType.DMA,
          pltpu.SemaphoreType.DMA,
      ],
      compiler_params=_CP,
  )
  def call_b(trm_hbm, it_hbm, out_hbm, ga0, ga1, tr0, tr1,
             iw0, iw1, iw2, iw3,
             sg0, sg1, st0, st1, sw0, sw1, sw2, sw3):
    wid = lax.axis_index("subcore") * 2 + lax.axis_index("core")
    gab = (ga0, ga1)
    trb = (tr0, tr1)
    iwb = (iw0, iw1, iw2, iw3)
    sg = (sg0, sg1)
    st = (st0, st1)
    sw = (sw0, sw1, sw2, sw3)

    def start_idx(w, p):
      h = lax.div(w, 4)
      bg = 4 * wid + lax.rem(w, 4)
      pltpu.async_copy(it_hbm.at[pl.ds(h, 1), pl.ds(bg * 128, 128)],
                       iwb[p], sw[p])

    def wait_idx(p):
      pltpu.make_async_copy(it_hbm.at[pl.ds(0, 1), pl.ds(0, 128)],
                            iwb[p], sw[p]).wait()

    def start_gather(p, s):
      pltpu.async_copy(trm_hbm.at[iwb[p].at[0]],
                       gab[s].at[pl.ds(0, 128), pl.ds(0, 128)], sg[s])

    def wait_gather(p, s):
      pltpu.make_async_copy(trm_hbm.at[iwb[p].at[0]],
                            gab[s].at[pl.ds(0, 128), pl.ds(0, 128)],
                            sg[s]).wait()

    def start_store(w, s):
      h = lax.div(w, 4)
      bg = 4 * wid + lax.rem(w, 4)
      pltpu.async_copy(trb[s],
                       out_hbm.at[h, pl.ds(0, 64), pl.ds(bg * 128, 128)],
                       st[s])

    def wait_store(s):
      pltpu.make_async_copy(trb[s],
                            out_hbm.at[0, pl.ds(0, 64), pl.ds(0, 128)],
                            st[s]).wait()

    it = _iota16()

    def transpose_window(s):
      @pl.loop(0, 8)
      def _(cb):
        rows = it + cb * 16
        for e4 in range(16):
          vals = []
          for j in range(4):
            e = e4 * 4 + j
            vals.append(plsc.load_gather(
                gab[s], [rows, jnp.full((16,), e, jnp.int32)]))
          for j in range(4):
            e = e4 * 4 + j
            trb[s][e, pl.ds(cb * 16, 16)] = vals[j]

    for p in range(4):
      start_idx(p, p)
    wait_idx(0)
    start_gather(0, 0)

    @pl.loop(0, 50)
    def _(w4):
      for ws in range(4):
        w = w4 * 4 + ws
        s = ws % 2
        wait_gather(ws, s)

        @pl.when(w + 1 < 200)
        def _():
          wait_idx((ws + 1) % 4)
          start_gather((ws + 1) % 4, (ws + 1) % 2)

        @pl.when(w >= 2)
        def _():
          wait_store(s)

        transpose_window(s)
        start_store(w, s)

        @pl.when(w + 4 < 200)
        def _():
          start_idx(w + 4, ws)

    for s in range(2):
      wait_store(s)

  return call_b(trm, idx_t)


def kernel(table, type_index):
  tail2 = table[999936:1000000].reshape(32, 128)
  x = _impl(table.T, tail2, type_index.T)
  return x.transpose(2, 0, 1)


# speedup vs baseline: 2.7525x; 1.4328x over previous
import functools

import jax
import jax.numpy as jnp
from jax.experimental import pallas as pl
from jax.experimental.pallas import tpu as pltpu
from jax.experimental.pallas import tpu_sc as plsc

_WINDOW = 512


@functools.partial(jax.jit, static_argnums=(2, 3))
def _sc_gather(table, idx_flat, n_idx, embed):
  mesh = plsc.VectorSubcoreMesh(core_axis_name="core",
                                subcore_axis_name="subcore")

  @functools.partial(
      pl.kernel,
      out_type=jax.ShapeDtypeStruct((n_idx, embed), table.dtype),
      mesh=mesh,
      compiler_params=pltpu.CompilerParams(use_tc_tiling_on_sc=False),
  )
  def gather_kernel(table_hbm, idx_hbm, out_hbm):
    def body(idx_vmem, out_vmem):
      pltpu.sync_copy(table_hbm.at[idx_vmem.at[0]], out_vmem)

    pltpu.emit_pipeline(
        body,
        grid=(n_idx // _WINDOW,),
        in_specs=[pl.BlockSpec((1, _WINDOW), index_map=lambda i: (0, i))],
        out_specs=[pl.BlockSpec((_WINDOW, embed), index_map=lambda i: (i, 0))],
        core_axis_name=("core", "subcore"),
        dimension_semantics=(pltpu.PARALLEL,),
    )(idx_hbm, out_hbm)

  return gather_kernel(table, idx_flat)


def kernel(table, type_index):
  batch, hist = type_index.shape
  embed = table.shape[1]
  n_idx = batch * hist
  idx_flat = type_index.reshape(1, n_idx)
  out = _sc_gather(table, idx_flat, n_idx, embed)
  return out.reshape(batch, hist, embed)
